# baseline (speedup 1.0000x reference)
import functools

import jax
import jax.numpy as jnp
from jax import lax
from jax.experimental import pallas as pl
from jax.experimental.pallas import tpu as pltpu
from jax.experimental.pallas import tpu_sc as plsc

F32 = jnp.float32
I32 = jnp.int32

N = 50000
E = 1600000
NB = 2048
N_PAD = 51200
W_EDGE = 1024
E_PAD = ((E + 32 * W_EDGE - 1) // (32 * W_EDGE)) * (32 * W_EDGE)
STRIPE = N_PAD // 16
NH = N_PAD // 2

_mesh = plsc.VectorSubcoreMesh(core_axis_name="c", subcore_axis_name="s")


def _t1_body(xt_ref, w_ref, as_ref, ad_ref, wh_ref, esed_ref):
    xb = xt_ref[...]
    whs = []
    rows = []
    for h in range(3):
        wh = jnp.dot(w_ref[h], xb, preferred_element_type=F32)
        whs.append(wh)
        rows.append(jnp.sum(wh * as_ref[h][:, None], axis=0))
    for h in range(3):
        rows.append(jnp.sum(whs[h] * ad_ref[h][:, None], axis=0))
    wh_ref[...] = jnp.concatenate(whs, axis=0)
    esed_ref[...] = jnp.stack(rows)


def _t1(xt, w1t, a1s, a1d):
    return pl.pallas_call(
        _t1_body,
        grid=(N_PAD // NB,),
        in_specs=[
            pl.BlockSpec((16, NB), lambda i: (0, i)),
            pl.BlockSpec((3, 16, 16), lambda i: (0, 0, 0)),
            pl.BlockSpec((3, 16), lambda i: (0, 0)),
            pl.BlockSpec((3, 16), lambda i: (0, 0)),
        ],
        out_specs=[
            pl.BlockSpec((48, NB), lambda i: (0, i)),
            pl.BlockSpec((6, NB), lambda i: (0, i)),
        ],
        out_shape=[
            jax.ShapeDtypeStruct((48, N_PAD), F32),
            jax.ShapeDtypeStruct((6, N_PAD), F32),
        ],
    )(xt, w1t, a1s, a1d)


def _t2_body(u_ref, d_ref, w_ref, as_ref, ad_ref, wh_ref, esed_ref):
    D = d_ref[0] + d_ref[1]
    h1 = []
    for hp in range(3):
        u = u_ref[0, hp] + u_ref[1, hp]
        v = u / (D[hp][None, :] + 1e-16)
        h1.append(jnp.where(v > 0, v, jnp.exp(v) - 1.0))
    rows = []
    accs = []
    for h in range(3):
        acc = jnp.zeros((64, NB), F32)
        for hp in range(3):
            acc = acc + jnp.dot(w_ref[h, hp], h1[hp],
                                preferred_element_type=F32)
        accs.append(acc)
        rows.append(jnp.sum(acc * as_ref[h][:, None], axis=0))
    for h in range(3):
        rows.append(jnp.sum(accs[h] * ad_ref[h][:, None], axis=0))
    wh_ref[...] = jnp.concatenate(accs, axis=0)
    esed_ref[...] = jnp.stack(rows)


def _t2(u1, den1, w2t, a2s, a2d):
    return pl.pallas_call(
        _t2_body,
        grid=(N_PAD // NB,),
        in_specs=[
            pl.BlockSpec((2, 3, 16, NB), lambda i: (0, 0, 0, i)),
            pl.BlockSpec((2, 3, NB), lambda i: (0, 0, i)),
            pl.BlockSpec((3, 3, 64, 16), lambda i: (0, 0, 0, 0)),
            pl.BlockSpec((3, 64), lambda i: (0, 0)),
            pl.BlockSpec((3, 64), lambda i: (0, 0)),
        ],
        out_specs=[
            pl.BlockSpec((192, NB), lambda i: (0, i)),
            pl.BlockSpec((6, NB), lambda i: (0, i)),
        ],
        out_shape=[
            jax.ShapeDtypeStruct((192, N_PAD), F32),
            jax.ShapeDtypeStruct((6, N_PAD), F32),
        ],
    )(u1, den1, w2t, a2s, a2d)


def _t3_body(u_ref, d_ref, wd1_ref, b1_ref, wd2_ref, b2_ref, s_ref, y_ref):
    i = pl.program_id(0)
    D = d_ref[0] + d_ref[1]
    parts = []
    for q in range(12):
        h = q // 4
        v = u_ref[q] / (D[h][None, :] + 1e-16)
        e = jnp.where(v > 0, v, jnp.exp(v) - 1.0)
        parts.append(jnp.sum(e, axis=1).reshape(1, 16))
    p = jnp.concatenate(parts, axis=1)

    @pl.when(i == 0)
    def _():
        s_ref[...] = p

    @pl.when(i > 0)
    def _():
        s_ref[...] = s_ref[...] + p

    s = s_ref[...]
    n = jnp.sqrt(jnp.sum(s * s))
    sn = s / jnp.maximum(n, 1e-12)
    hm = jnp.maximum(jnp.dot(sn, wd1_ref[...], preferred_element_type=F32)
                     + b1_ref[...], 0.0)
    y_ref[...] = jnp.dot(hm, wd2_ref[...], preferred_element_type=F32) + b2_ref[...]


def _t3(u2, den2, wd1p, b1, wd2p, b2p):
    return pl.pallas_call(
        _t3_body,
        grid=(N_PAD // NB,),
        in_specs=[
            pl.BlockSpec((12, 16, NB), lambda i: (0, 0, i)),
            pl.BlockSpec((2, 3, NB), lambda i: (0, 0, i)),
            pl.BlockSpec((192, 128), lambda i: (0, 0)),
            pl.BlockSpec((1, 128), lambda i: (0, 0)),
            pl.BlockSpec((128, 128), lambda i: (0, 0)),
            pl.BlockSpec((1, 128), lambda i: (0, 0)),
        ],
        out_specs=[
            pl.BlockSpec((1, 192), lambda i: (0, 0)),
            pl.BlockSpec((1, 128), lambda i: (0, 0)),
        ],
        out_shape=[
            jax.ShapeDtypeStruct((1, 192), F32),
            jax.ShapeDtypeStruct((1, 128), F32),
        ],
    )(u2, den2, wd1p, b1, wd2p, b2p)


@functools.partial(
    pl.kernel,
    mesh=_mesh,
    compiler_params=pltpu.CompilerParams(needs_layout_passes=False),
    out_type=[
        jax.ShapeDtypeStruct((3 * E_PAD,), F32),
        jax.ShapeDtypeStruct((6 * N_PAD,), F32),
    ],
    scratch_types=[
        pltpu.VMEM((N_PAD,), F32),
        pltpu.VMEM((N_PAD,), F32),
        pltpu.VMEM((8, 128), I32),
        pltpu.VMEM((8, 128), I32),
        pltpu.VMEM((W_EDGE,), F32),
        pltpu.VMEM((STRIPE,), F32),
        pltpu.VMEM_SHARED((N_PAD,), F32),
    ],
)
def _s1(esed, src2, dst2, z_out, den_out, es_t, ed_t, src_t, dst_t, z_t,
        zero_t, den_sp):
    cid = lax.axis_index("c")
    sid = lax.axis_index("s")
    tpt = E_PAD // 32
    tile_base = cid * (E_PAD // 2) + sid * tpt
    nw = tpt // W_EDGE

    def zf(i, c):
        zero_t[pl.ds(i * 16, 16)] = jnp.zeros((16,), F32)
        return c
    lax.fori_loop(0, STRIPE // 16, zf, 0)

    for h in range(3):
        pltpu.sync_copy(esed.at[pl.ds(h * N_PAD, N_PAD)], es_t)
        pltpu.sync_copy(esed.at[pl.ds((3 + h) * N_PAD, N_PAD)], ed_t)
        pltpu.sync_copy(zero_t,
                        den_sp.at[pl.ds(pl.multiple_of(sid * STRIPE, 8), STRIPE)])
        plsc.subcore_barrier()

        def wbody(w, c):
            base = pl.multiple_of(tile_base + w * W_EDGE, 1024)
            brow = pl.multiple_of(base // 128, 8)
            pltpu.sync_copy(src2.at[pl.ds(brow, 8)], src_t)
            pltpu.sync_copy(dst2.at[pl.ds(brow, 8)], dst_t)
            for k in range(W_EDGE // 16):
                j, kk = k // 8, (k % 8) * 16
                si = src_t[j, pl.ds(kk, 16)]
                di = dst_t[j, pl.ds(kk, 16)]
                e = plsc.load_gather(es_t, [si]) + plsc.load_gather(ed_t, [di])
                e = jnp.where(e > 0.0, e, 0.2 * e)
                zv = jnp.exp(jnp.minimum(e, 50.0))
                gid = base + (k * 16) + lax.iota(I32, 16)
                z_t[pl.ds(k * 16, 16)] = jnp.where(gid < E, zv, 0.0)
            pltpu.sync_copy(z_t, z_out.at[pl.ds(h * E_PAD + base, W_EDGE)])
            for j in range(8):
                pltpu.sync_copy(z_t.at[pl.ds(j * 128, 128)],
                                den_sp.at[dst_t.at[j]], add=True)
            return c
        lax.fori_loop(0, nw, wbody, 0)
        plsc.subcore_barrier()

        @pl.when(sid == 0)
        def _():
            off = pl.multiple_of((3 * cid + h) * N_PAD, 128)
            pltpu.sync_copy(den_sp, den_out.at[pl.ds(off, N_PAD)])
        plsc.subcore_barrier()


def _make_s2(nchunk, feat_split, WS):
    chunks_per_core = nchunk // 2 if feat_split else nchunk
    heads_per = nchunk // 3
    out_elems = (nchunk if feat_split else 2 * nchunk) * 16 * N_PAD

    @functools.partial(
        pl.kernel,
        mesh=_mesh,
        compiler_params=pltpu.CompilerParams(needs_layout_passes=False),
        out_type=jax.ShapeDtypeStruct((out_elems,), F32),
        scratch_types=[
            pltpu.VMEM((WS,), I32),
            pltpu.VMEM((WS,), I32),
            pltpu.VMEM((WS,), I32),
            pltpu.VMEM((WS,), F32),
            pltpu.VMEM((WS,), F32),
            pltpu.VMEM((16 * WS,), F32),
            pltpu.VMEM((STRIPE,), F32),
            pltpu.VMEM_SHARED((16 * N_PAD,), F32),
            pltpu.VMEM_SHARED((16 * NH,), F32),
            pltpu.SemaphoreType.DMA,
            pltpu.SemaphoreType.DMA,
        ],
    )
    def s2(wh, src1, dst1, z, u_out, src_t, dst_t, dst_c, z_t, z_m, vals_t,
           zero_t, tab_sp, u_sp, gsem, ssem):
        cid = lax.axis_index("c")
        sid = lax.axis_index("s")
        if feat_split:
            tpt = E_PAD // 16
            tile_base = sid * tpt
        else:
            tpt = E_PAD // 32
            tile_base = cid * (E_PAD // 2) + sid * tpt
        nw = tpt // WS

        def zf(i, c):
            zero_t[pl.ds(i * 16, 16)] = jnp.zeros((16,), F32)
            return c
        lax.fori_loop(0, STRIPE // 16, zf, 0)

        for ci in range(chunks_per_core):
            if feat_split:
                chunk = chunks_per_core * cid + ci
            else:
                chunk = ci
            hz = chunk // heads_per
            coff = pl.multiple_of(chunk * 16 * N_PAD, 128)

            @pl.when(sid == 0)
            def _():
                pltpu.sync_copy(wh.at[pl.ds(coff, 16 * N_PAD)], tab_sp)

            for hp in range(2):
                half_base = hp * NH

                def zc(r, c):
                    roff = pl.multiple_of(sid * (16 * NH // 16) + r * STRIPE, 8)
                    pltpu.sync_copy(zero_t, u_sp.at[pl.ds(roff, STRIPE)])
                    return c
                lax.fori_loop(0, (16 * NH // 16) // STRIPE, zc, 0)
                plsc.subcore_barrier()

                def wbody(w, c):
                    base = pl.multiple_of(tile_base + w * WS, WS)
                    pltpu.sync_copy(src1.at[pl.ds(base, WS)], src_t)
                    pltpu.sync_copy(dst1.at[pl.ds(base, WS)], dst_t)
                    zoff = pl.multiple_of(hz * E_PAD + base, WS)
                    pltpu.sync_copy(z.at[pl.ds(zoff, WS)], z_t)
                    cps = [pltpu.async_copy(
                        tab_sp.at[pl.ds(pl.multiple_of(cc * N_PAD, 128),
                                        N_PAD)].at[src_t],
                        vals_t.at[pl.ds(cc * WS, WS)], gsem)
                        for cc in range(16)]
                    for k in range(WS // 16):
                        sl = pl.ds(k * 16, 16)
                        rel = dst_t[sl] - half_base
                        m = jnp.logical_and(rel >= 0, rel < NH)
                        z_m[sl] = jnp.where(m, z_t[sl], 0.0)
                        dst_c[sl] = jnp.where(m, rel, 0)
                    for cp in cps:
                        cp.wait()

                    def ebody(k, c2):
                        zv = z_m[pl.ds(k * 16, 16)]
                        for cc in range(16):
                            sl = pl.ds(cc * WS + k * 16, 16)
                            vals_t[sl] = vals_t[sl] * zv
                        return c2
                    lax.fori_loop(0, WS // 16, ebody, 0)
                    sps = [pltpu.async_copy(
                        vals_t.at[pl.ds(cc * WS, WS)],
                        u_sp.at[pl.ds(pl.multiple_of(cc * NH, 128),
                                      NH)].at[dst_c],
                        ssem, add=True)
                        for cc in range(16)]
                    for sp in sps:
                        sp.wait()
                    return c
                lax.fori_loop(0, nw, wbody, 0)
                plsc.subcore_barrier()

                if feat_split:
                    uoff0 = coff
                else:
                    uoff0 = pl.multiple_of(
                        (cid * chunks_per_core + ci) * 16 * N_PAD, 128)

                @pl.when(sid == 0)
                def _():
                    for cc in range(16):
                        uoff = pl.multiple_of(
                            uoff0 + cc * N_PAD + half_base, 128)
                        pltpu.sync_copy(
                            u_sp.at[pl.ds(pl.multiple_of(cc * NH, 128), NH)],
                            u_out.at[pl.ds(uoff, NH)])
                plsc.subcore_barrier()

    return s2


_s2_l1 = _make_s2(3, False, 1024)
_s2_l2 = _make_s2(12, True, 2048)


def kernel(x, edge_index, W1, a1s, a1d, W2, a2s, a2d, Wd1, bd1, Wd2, bd2):
    xt = jnp.pad(x, ((0, N_PAD - N), (0, 16 - x.shape[1]))).T
    w1t = jnp.transpose(jnp.pad(W1, ((0, 0), (0, 16 - W1.shape[1]), (0, 0))),
                        (0, 2, 1))
    w2t = jnp.transpose(W2.reshape(3, 3, 16, 64), (0, 1, 3, 2))
    src = edge_index[0].astype(I32)
    dst = edge_index[1].astype(I32)
    P = E_PAD - E
    pad_idx = N + (jnp.arange(P, dtype=I32) % 1024)
    src1 = jnp.concatenate([src, pad_idx])
    dst1 = jnp.concatenate([dst, pad_idx])
    src2 = src1.reshape(E_PAD // 128, 128)
    dst2 = dst1.reshape(E_PAD // 128, 128)
    perm = jnp.array([64 * (q // 4) + 16 * (q % 4) + j
                      for q in range(12) for j in range(16)], dtype=I32)
    wd1p = Wd1[perm, :]
    b1 = bd1.reshape(1, 128)
    wd2p = jnp.pad(Wd2, ((0, 0), (0, 127)))
    b2p = jnp.pad(bd2.reshape(1, 1), ((0, 0), (0, 127)))

    wh1, esed1 = _t1(xt, w1t, a1s, a1d)
    z1, den1 = _s1(esed1.reshape(-1), src2, dst2)
    den1 = den1.reshape(2, 3, N_PAD)
    u1 = _s2_l1(wh1.reshape(-1), src1, dst1, z1)
    u1 = u1.reshape(2, 3, 16, N_PAD)
    wh2, esed2 = _t2(u1, den1, w2t, a2s, a2d)
    z2, den2 = _s1(esed2.reshape(-1), src2, dst2)
    den2 = den2.reshape(2, 3, N_PAD)
    u2 = _s2_l2(wh2.reshape(-1), src1, dst1, z2)
    u2 = u2.reshape(12, 16, N_PAD)
    _, y = _t3(u2, den2, wd1p, b1, wd2p, b2p)
    return y[0, 0:1]


# speedup vs baseline: 1.0665x; 1.0665x over previous
import functools

import jax
import jax.numpy as jnp
from jax import lax
from jax.experimental import pallas as pl
from jax.experimental.pallas import tpu as pltpu
from jax.experimental.pallas import tpu_sc as plsc

F32 = jnp.float32
I32 = jnp.int32

N = 50000
E = 1600000
NB = 2048
N_PAD = 51200
W_EDGE = 1024
E_PAD = ((E + 32 * W_EDGE - 1) // (32 * W_EDGE)) * (32 * W_EDGE)
STRIPE = N_PAD // 16
NH = N_PAD // 2

_mesh = plsc.VectorSubcoreMesh(core_axis_name="c", subcore_axis_name="s")


def _t1_body(xt_ref, w_ref, as_ref, ad_ref, wh_ref, esed_ref):
    xb = xt_ref[...]
    whs = []
    rows = []
    for h in range(3):
        wh = jnp.dot(w_ref[h], xb, preferred_element_type=F32)
        whs.append(wh)
        rows.append(jnp.sum(wh * as_ref[h][:, None], axis=0))
    for h in range(3):
        rows.append(jnp.sum(whs[h] * ad_ref[h][:, None], axis=0))
    wh_ref[...] = jnp.concatenate(whs, axis=0)
    esed_ref[...] = jnp.stack(rows)


def _t1(xt, w1t, a1s, a1d):
    return pl.pallas_call(
        _t1_body,
        grid=(N_PAD // NB,),
        in_specs=[
            pl.BlockSpec((16, NB), lambda i: (0, i)),
            pl.BlockSpec((3, 16, 16), lambda i: (0, 0, 0)),
            pl.BlockSpec((3, 16), lambda i: (0, 0)),
            pl.BlockSpec((3, 16), lambda i: (0, 0)),
        ],
        out_specs=[
            pl.BlockSpec((48, NB), lambda i: (0, i)),
            pl.BlockSpec((6, NB), lambda i: (0, i)),
        ],
        out_shape=[
            jax.ShapeDtypeStruct((48, N_PAD), F32),
            jax.ShapeDtypeStruct((6, N_PAD), F32),
        ],
    )(xt, w1t, a1s, a1d)


def _t2_body(u_ref, d_ref, w_ref, as_ref, ad_ref, wh_ref, esed_ref):
    D = d_ref[0] + d_ref[1]
    h1 = []
    for hp in range(3):
        u = u_ref[0, hp] + u_ref[1, hp]
        v = u / (D[hp][None, :] + 1e-16)
        h1.append(jnp.where(v > 0, v, jnp.exp(v) - 1.0))
    rows = []
    accs = []
    for h in range(3):
        acc = jnp.zeros((64, NB), F32)
        for hp in range(3):
            acc = acc + jnp.dot(w_ref[h, hp], h1[hp],
                                preferred_element_type=F32)
        accs.append(acc)
        rows.append(jnp.sum(acc * as_ref[h][:, None], axis=0))
    for h in range(3):
        rows.append(jnp.sum(accs[h] * ad_ref[h][:, None], axis=0))
    wh_ref[...] = jnp.concatenate(accs, axis=0)
    esed_ref[...] = jnp.stack(rows)


def _t2(u1, den1, w2t, a2s, a2d):
    return pl.pallas_call(
        _t2_body,
        grid=(N_PAD // NB,),
        in_specs=[
            pl.BlockSpec((2, 3, 16, NB), lambda i: (0, 0, 0, i)),
            pl.BlockSpec((2, 3, NB), lambda i: (0, 0, i)),
            pl.BlockSpec((3, 3, 64, 16), lambda i: (0, 0, 0, 0)),
            pl.BlockSpec((3, 64), lambda i: (0, 0)),
            pl.BlockSpec((3, 64), lambda i: (0, 0)),
        ],
        out_specs=[
            pl.BlockSpec((192, NB), lambda i: (0, i)),
            pl.BlockSpec((6, NB), lambda i: (0, i)),
        ],
        out_shape=[
            jax.ShapeDtypeStruct((192, N_PAD), F32),
            jax.ShapeDtypeStruct((6, N_PAD), F32),
        ],
    )(u1, den1, w2t, a2s, a2d)


def _t3_body(u_ref, d_ref, wd1_ref, b1_ref, wd2_ref, b2_ref, s_ref, y_ref):
    i = pl.program_id(0)
    D = d_ref[0] + d_ref[1]
    parts = []
    for q in range(12):
        h = q // 4
        v = u_ref[q] / (D[h][None, :] + 1e-16)
        e = jnp.where(v > 0, v, jnp.exp(v) - 1.0)
        parts.append(jnp.sum(e, axis=1).reshape(1, 16))
    p = jnp.concatenate(parts, axis=1)

    @pl.when(i == 0)
    def _():
        s_ref[...] = p

    @pl.when(i > 0)
    def _():
        s_ref[...] = s_ref[...] + p

    s = s_ref[...]
    n = jnp.sqrt(jnp.sum(s * s))
    sn = s / jnp.maximum(n, 1e-12)
    hm = jnp.maximum(jnp.dot(sn, wd1_ref[...], preferred_element_type=F32)
                     + b1_ref[...], 0.0)
    y_ref[...] = jnp.dot(hm, wd2_ref[...], preferred_element_type=F32) + b2_ref[...]


def _t3(u2, den2, wd1p, b1, wd2p, b2p):
    return pl.pallas_call(
        _t3_body,
        grid=(N_PAD // NB,),
        in_specs=[
            pl.BlockSpec((12, 16, NB), lambda i: (0, 0, i)),
            pl.BlockSpec((2, 3, NB), lambda i: (0, 0, i)),
            pl.BlockSpec((192, 128), lambda i: (0, 0)),
            pl.BlockSpec((1, 128), lambda i: (0, 0)),
            pl.BlockSpec((128, 128), lambda i: (0, 0)),
            pl.BlockSpec((1, 128), lambda i: (0, 0)),
        ],
        out_specs=[
            pl.BlockSpec((1, 192), lambda i: (0, 0)),
            pl.BlockSpec((1, 128), lambda i: (0, 0)),
        ],
        out_shape=[
            jax.ShapeDtypeStruct((1, 192), F32),
            jax.ShapeDtypeStruct((1, 128), F32),
        ],
    )(u2, den2, wd1p, b1, wd2p, b2p)


@functools.partial(
    pl.kernel,
    mesh=_mesh,
    compiler_params=pltpu.CompilerParams(needs_layout_passes=False),
    out_type=[
        jax.ShapeDtypeStruct((3 * E_PAD,), F32),
        jax.ShapeDtypeStruct((6 * N_PAD,), F32),
    ],
    scratch_types=[
        pltpu.VMEM((N_PAD,), F32),
        pltpu.VMEM((N_PAD,), F32),
        pltpu.VMEM((8, 128), I32),
        pltpu.VMEM((8, 128), I32),
        pltpu.VMEM((W_EDGE,), F32),
        pltpu.VMEM((STRIPE,), F32),
        pltpu.VMEM_SHARED((N_PAD,), F32),
    ],
)
def _s1(esed, src2, dst2, z_out, den_out, es_t, ed_t, src_t, dst_t, z_t,
        zero_t, den_sp):
    cid = lax.axis_index("c")
    sid = lax.axis_index("s")
    tpt = E_PAD // 32
    tile_base = cid * (E_PAD // 2) + sid * tpt
    nw = tpt // W_EDGE

    def zf(i, c):
        zero_t[pl.ds(i * 16, 16)] = jnp.zeros((16,), F32)
        return c
    lax.fori_loop(0, STRIPE // 16, zf, 0)

    for h in range(3):
        pltpu.sync_copy(esed.at[pl.ds(h * N_PAD, N_PAD)], es_t)
        pltpu.sync_copy(esed.at[pl.ds((3 + h) * N_PAD, N_PAD)], ed_t)
        pltpu.sync_copy(zero_t,
                        den_sp.at[pl.ds(pl.multiple_of(sid * STRIPE, 8), STRIPE)])
        plsc.subcore_barrier()

        def wbody(w, c):
            base = pl.multiple_of(tile_base + w * W_EDGE, 1024)
            brow = pl.multiple_of(base // 128, 8)
            pltpu.sync_copy(src2.at[pl.ds(brow, 8)], src_t)
            pltpu.sync_copy(dst2.at[pl.ds(brow, 8)], dst_t)
            for k in range(W_EDGE // 16):
                j, kk = k // 8, (k % 8) * 16
                si = src_t[j, pl.ds(kk, 16)]
                di = dst_t[j, pl.ds(kk, 16)]
                e = plsc.load_gather(es_t, [si]) + plsc.load_gather(ed_t, [di])
                e = jnp.where(e > 0.0, e, 0.2 * e)
                zv = jnp.exp(jnp.minimum(e, 50.0))
                gid = base + (k * 16) + lax.iota(I32, 16)
                z_t[pl.ds(k * 16, 16)] = jnp.where(gid < E, zv, 0.0)
            pltpu.sync_copy(z_t, z_out.at[pl.ds(h * E_PAD + base, W_EDGE)])
            for j in range(8):
                pltpu.sync_copy(z_t.at[pl.ds(j * 128, 128)],
                                den_sp.at[dst_t.at[j]], add=True)
            return c
        lax.fori_loop(0, nw, wbody, 0)
        plsc.subcore_barrier()

        @pl.when(sid == 0)
        def _():
            off = pl.multiple_of((3 * cid + h) * N_PAD, 128)
            pltpu.sync_copy(den_sp, den_out.at[pl.ds(off, N_PAD)])
        plsc.subcore_barrier()


def _make_s2(nchunk, feat_split, WS):
    chunks_per_core = nchunk // 2 if feat_split else nchunk
    heads_per = nchunk // 3
    out_elems = (nchunk if feat_split else 2 * nchunk) * 16 * N_PAD

    @functools.partial(
        pl.kernel,
        mesh=_mesh,
        compiler_params=pltpu.CompilerParams(needs_layout_passes=False),
        out_type=jax.ShapeDtypeStruct((out_elems,), F32),
        scratch_types=[
            pltpu.VMEM((WS,), I32),
            pltpu.VMEM((WS,), I32),
            pltpu.VMEM((WS,), I32),
            pltpu.VMEM((WS,), F32),
            pltpu.VMEM((WS,), F32),
            pltpu.VMEM((16 * WS,), F32),
            pltpu.VMEM((STRIPE,), F32),
            pltpu.VMEM_SHARED((16 * N_PAD,), F32),
            pltpu.VMEM_SHARED((16 * NH,), F32),
            pltpu.SemaphoreType.DMA,
            pltpu.SemaphoreType.DMA,
        ],
    )
    def s2(wh, src1, dst1, z, u_out, src_t, dst_t, dst_c, z_t, z_m, vals_t,
           zero_t, tab_sp, u_sp, gsem, ssem):
        cid = lax.axis_index("c")
        sid = lax.axis_index("s")
        if feat_split:
            tpt = E_PAD // 16
            tile_base = sid * tpt
        else:
            tpt = E_PAD // 32
            tile_base = cid * (E_PAD // 2) + sid * tpt
        nw = tpt // WS

        def zf(i, c):
            zero_t[pl.ds(i * 16, 16)] = jnp.zeros((16,), F32)
            return c
        lax.fori_loop(0, STRIPE // 16, zf, 0)

        for ci in range(chunks_per_core):
            if feat_split:
                chunk = chunks_per_core * cid + ci
            else:
                chunk = ci
            hz = chunk // heads_per
            coff = pl.multiple_of(chunk * 16 * N_PAD, 128)

            @pl.when(sid == 0)
            def _():
                pltpu.sync_copy(wh.at[pl.ds(coff, 16 * N_PAD)], tab_sp)

            for hp in range(2):
                half_base = hp * NH

                def zc(r, c):
                    roff = pl.multiple_of(sid * (16 * NH // 16) + r * STRIPE, 8)
                    pltpu.sync_copy(zero_t, u_sp.at[pl.ds(roff, STRIPE)])
                    return c
                lax.fori_loop(0, (16 * NH // 16) // STRIPE, zc, 0)
                plsc.subcore_barrier()

                def wbody(w, c):
                    base = pl.multiple_of(tile_base + w * WS, WS)
                    pltpu.sync_copy(src1.at[pl.ds(base, WS)], src_t)
                    pltpu.sync_copy(dst1.at[pl.ds(base, WS)], dst_t)
                    zoff = pl.multiple_of(hz * E_PAD + base, WS)
                    pltpu.sync_copy(z.at[pl.ds(zoff, WS)], z_t)
                    cps = [pltpu.async_copy(
                        tab_sp.at[pl.ds(pl.multiple_of(cc * N_PAD, 128),
                                        N_PAD)].at[src_t],
                        vals_t.at[pl.ds(cc * WS, WS)], gsem)
                        for cc in range(16)]
                    for k in range(WS // 16):
                        sl = pl.ds(k * 16, 16)
                        rel = dst_t[sl] - half_base
                        m = jnp.logical_and(rel >= 0, rel < NH)
                        z_m[sl] = jnp.where(m, z_t[sl], 0.0)
                        dst_c[sl] = jnp.where(m, rel, 0)
                    for cp in cps:
                        cp.wait()

                    def ebody(k, c2):
                        zv = z_m[pl.ds(k * 16, 16)]
                        for cc in range(16):
                            sl = pl.ds(cc * WS + k * 16, 16)
                            vals_t[sl] = vals_t[sl] * zv
                        return c2
                    lax.fori_loop(0, WS // 16, ebody, 0)
                    sps = [pltpu.async_copy(
                        vals_t.at[pl.ds(cc * WS, WS)],
                        u_sp.at[pl.ds(pl.multiple_of(cc * NH, 128),
                                      NH)].at[dst_c],
                        ssem, add=True)
                        for cc in range(16)]
                    for sp in sps:
                        sp.wait()
                    return c
                lax.fori_loop(0, nw, wbody, 0)
                plsc.subcore_barrier()

                if feat_split:
                    uoff0 = coff
                else:
                    uoff0 = pl.multiple_of(
                        (cid * chunks_per_core + ci) * 16 * N_PAD, 128)

                @pl.when(sid == 0)
                def _():
                    for cc in range(16):
                        uoff = pl.multiple_of(
                            uoff0 + cc * N_PAD + half_base, 128)
                        pltpu.sync_copy(
                            u_sp.at[pl.ds(pl.multiple_of(cc * NH, 128), NH)],
                            u_out.at[pl.ds(uoff, NH)])
                plsc.subcore_barrier()

    return s2


_s2_l1 = _make_s2(3, False, 512)
_s2_l2 = _make_s2(12, True, 512)


def kernel(x, edge_index, W1, a1s, a1d, W2, a2s, a2d, Wd1, bd1, Wd2, bd2):
    xt = jnp.pad(x, ((0, N_PAD - N), (0, 16 - x.shape[1]))).T
    w1t = jnp.transpose(jnp.pad(W1, ((0, 0), (0, 16 - W1.shape[1]), (0, 0))),
                        (0, 2, 1))
    w2t = jnp.transpose(W2.reshape(3, 3, 16, 64), (0, 1, 3, 2))
    src = edge_index[0].astype(I32)
    dst = edge_index[1].astype(I32)
    P = E_PAD - E
    pad_idx = N + (jnp.arange(P, dtype=I32) % 1024)
    src1 = jnp.concatenate([src, pad_idx])
    dst1 = jnp.concatenate([dst, pad_idx])
    src2 = src1.reshape(E_PAD // 128, 128)
    dst2 = dst1.reshape(E_PAD // 128, 128)
    perm = jnp.array([64 * (q // 4) + 16 * (q % 4) + j
                      for q in range(12) for j in range(16)], dtype=I32)
    wd1p = Wd1[perm, :]
    b1 = bd1.reshape(1, 128)
    wd2p = jnp.pad(Wd2, ((0, 0), (0, 127)))
    b2p = jnp.pad(bd2.reshape(1, 1), ((0, 0), (0, 127)))

    wh1, esed1 = _t1(xt, w1t, a1s, a1d)
    z1, den1 = _s1(esed1.reshape(-1), src2, dst2)
    den1 = den1.reshape(2, 3, N_PAD)
    u1 = _s2_l1(wh1.reshape(-1), src1, dst1, z1)
    u1 = u1.reshape(2, 3, 16, N_PAD)
    wh2, esed2 = _t2(u1, den1, w2t, a2s, a2d)
    z2, den2 = _s1(esed2.reshape(-1), src2, dst2)
    den2 = den2.reshape(2, 3, N_PAD)
    u2 = _s2_l2(wh2.reshape(-1), src1, dst1, z2)
    u2 = u2.reshape(12, 16, N_PAD)
    _, y = _t3(u2, den2, wd1p, b1, wd2p, b2p)
    return y[0, 0:1]
